# baseline (speedup 1.0000x reference)
import jax
import jax.numpy as jnp
from jax.experimental import pallas as pl
from jax.experimental.pallas import tpu as pltpu

N = 10000
H = 32
D = 128
K = 15
C = 13
KP_EXTENT = 1.2

B = 400
NB = N // B


def _enc_body(kp_ref, p_ref, pz_ref,
              npx_ref, npy_ref, npz_ref, nf_ref,
              Wkp_ref, We1_ref, We2_ref, Wq_ref, Wk_ref,
              x_ref, q_ref, kk_ref, energy_ref, acc_ref):
    i = pl.program_id(0)
    dx = npx_ref[...] - p_ref[:, 0:1]
    dy = npy_ref[...] - p_ref[:, 1:2]
    dz = npz_ref[...] - p_ref[:, 2:3]
    n2 = dx * dx + dy * dy + dz * dz
    nf = nf_ref[...]
    x = jnp.zeros((B, D), jnp.float32)
    for k in range(K):
        kpx = kp_ref[k, 0]
        kpy = kp_ref[k, 1]
        kpz = kp_ref[k, 2]
        kp2 = kpx * kpx + kpy * kpy + kpz * kpz
        d2 = n2 - 2.0 * (dx * kpx + dy * kpy + dz * kpz) + kp2
        infl = jnp.maximum(0.0, 1.0 - jnp.sqrt(d2 + 1e-12) / KP_EXTENT)
        wk = jnp.sum(infl[:, :, None] * nf, axis=1)
        x = x + jnp.dot(wk, Wkp_ref[k], preferred_element_type=jnp.float32)
    x = jnp.maximum(x, 0.0)
    ele = pz_ref[...]
    h1 = jnp.maximum(ele * We1_ref[...], 0.0)
    gate = jax.nn.sigmoid(jnp.dot(h1, We2_ref[...],
                                  preferred_element_type=jnp.float32))
    x = x * gate
    x_ref[...] = x
    q_ref[...] = jnp.dot(x, Wq_ref[...], preferred_element_type=jnp.float32)
    kk_ref[...] = jnp.dot(x, Wk_ref[...], preferred_element_type=jnp.float32)

    @pl.when(i == 0)
    def _():
        acc_ref[...] = jnp.zeros((D, D), jnp.float32)

    acc_ref[...] += jax.lax.dot_general(
        x, x, (((0,), (0,)), ((), ())), preferred_element_type=jnp.float32)

    @pl.when(i == NB - 1)
    def _():
        energy_ref[...] = acc_ref[...]


def _softmax_body(energy_ref, A_ref):
    e = energy_ref[...] * (1.0 / N)
    e = e - jnp.max(e, axis=-1, keepdims=True)
    ee = jnp.exp(e)
    A_ref[...] = ee / jnp.sum(ee, axis=-1, keepdims=True)


def _spa_body(x_ref, q_ref, kkgT_ref, xg_ref, spa_ref):
    x = x_ref[...]
    q = q_ref[...]
    s = jnp.zeros((B, H), jnp.float32)
    for d in range(32):
        s = s + q[:, d:d + 1] * kkgT_ref[:, d, :]
    s = s * (1.0 / jnp.sqrt(32.0))
    s = s - jnp.max(s, axis=-1, keepdims=True)
    es = jnp.exp(s)
    a = es / jnp.sum(es, axis=-1, keepdims=True)
    spa_ref[...] = jnp.sum(a[:, :, None] * xg_ref[...], axis=1) + x


def _att_body(x_ref, spa_ref, A_ref,
              Wpoi_ref, Whead_ref, Wdec_ref,
              out_ref, c0_ref, c1_ref, c2_ref, c3_ref,
              p0_ref, p1_ref, p2_ref, p3_ref):
    x = x_ref[...]
    spa = spa_ref[...]
    cha = jnp.dot(x, A_ref[...], preferred_element_type=jnp.float32) + x
    poi = x * jax.nn.sigmoid(jnp.dot(x, Wpoi_ref[...],
                                     preferred_element_type=jnp.float32))
    Whead = Whead_ref[...]
    Wdec = Wdec_ref[...]
    l0 = jnp.dot(x, Whead, preferred_element_type=jnp.float32)
    l1 = jnp.dot(poi, Whead, preferred_element_type=jnp.float32)
    l2 = jnp.dot(spa, Whead, preferred_element_type=jnp.float32)
    l3 = jnp.dot(cha, Whead, preferred_element_type=jnp.float32)

    ones = jnp.ones((B, 1), jnp.float32)
    p0_ref[0] = jax.lax.dot_general(ones, l0, (((0,), (0,)), ((), ())),
                                    preferred_element_type=jnp.float32)
    p1_ref[0] = jax.lax.dot_general(ones, l1, (((0,), (0,)), ((), ())),
                                    preferred_element_type=jnp.float32)
    p2_ref[0] = jax.lax.dot_general(ones, l2, (((0,), (0,)), ((), ())),
                                    preferred_element_type=jnp.float32)
    p3_ref[0] = jax.lax.dot_general(ones, l3, (((0,), (0,)), ((), ())),
                                    preferred_element_type=jnp.float32)

    c0 = jnp.maximum(jnp.dot(l0, Wdec, preferred_element_type=jnp.float32), 0.0)
    c1 = jnp.maximum(jnp.dot(l1, Wdec, preferred_element_type=jnp.float32), 0.0)
    c2 = jnp.maximum(jnp.dot(l2, Wdec, preferred_element_type=jnp.float32), 0.0)
    c3 = jnp.maximum(jnp.dot(l3, Wdec, preferred_element_type=jnp.float32), 0.0)
    c0_ref[...] = c0
    c1_ref[...] = c1
    c2_ref[...] = c2
    c3_ref[...] = c3
    out_ref[...] = jnp.maximum(jnp.maximum(c0, c1), jnp.maximum(c2, c3))


def _row_spec(last=None):
    if last is None:
        return pl.BlockSpec((B, H), lambda i: (i, 0))
    return pl.BlockSpec((B, H, last), lambda i: (i, 0, 0))


def _full(shape):
    nd = len(shape)
    return pl.BlockSpec(shape, lambda i: (0,) * nd)


def kernel(features, points, neighbors, kernel_points, W_kp,
           W_ele1, W_ele2, Wq, Wk, W_poi, W_head, W_dec):
    neighbors = neighbors.astype(jnp.int32)
    px = points[:, 0]
    py = points[:, 1]
    pz = points[:, 2]
    npx = px[neighbors]
    npy = py[neighbors]
    npz = pz[neighbors]
    nf = features[neighbors]

    x, q, kk, energy = pl.pallas_call(
        _enc_body,
        grid=(NB,),
        in_specs=[
            pl.BlockSpec(memory_space=pltpu.SMEM),
            pl.BlockSpec((B, 3), lambda i: (i, 0)),
            pl.BlockSpec((B, 1), lambda i: (i, 0)),
            _row_spec(), _row_spec(), _row_spec(),
            _row_spec(D),
            _full((K, D, D)),
            _full((1, 32)), _full((32, D)),
            _full((D, 32)), _full((D, 32)),
        ],
        out_specs=[
            pl.BlockSpec((B, D), lambda i: (i, 0)),
            pl.BlockSpec((B, 32), lambda i: (i, 0)),
            pl.BlockSpec((B, 32), lambda i: (i, 0)),
            pl.BlockSpec((D, D), lambda i: (0, 0)),
        ],
        out_shape=[
            jax.ShapeDtypeStruct((N, D), jnp.float32),
            jax.ShapeDtypeStruct((N, 32), jnp.float32),
            jax.ShapeDtypeStruct((N, 32), jnp.float32),
            jax.ShapeDtypeStruct((D, D), jnp.float32),
        ],
        scratch_shapes=[pltpu.VMEM((D, D), jnp.float32)],
    )(kernel_points, points, pz[:, None], npx, npy, npz, nf,
      W_kp, W_ele1, W_ele2, Wq, Wk)

    A = pl.pallas_call(
        _softmax_body,
        out_shape=jax.ShapeDtypeStruct((D, D), jnp.float32),
    )(energy)

    kkgT = jnp.swapaxes(kk[neighbors], 1, 2)
    xg = x[neighbors]

    spa = pl.pallas_call(
        _spa_body,
        grid=(NB,),
        in_specs=[
            pl.BlockSpec((B, D), lambda i: (i, 0)),
            pl.BlockSpec((B, 32), lambda i: (i, 0)),
            pl.BlockSpec((B, 32, H), lambda i: (i, 0, 0)),
            _row_spec(D),
        ],
        out_specs=pl.BlockSpec((B, D), lambda i: (i, 0)),
        out_shape=jax.ShapeDtypeStruct((N, D), jnp.float32),
    )(x, q, kkgT, xg)

    Whead_p = jnp.pad(W_head, ((0, 0), (0, 128 - C)))
    Wdec_p = jnp.pad(W_dec, ((0, 128 - C), (0, 128 - C)))

    out_p, c0, c1, c2, c3, p0, p1, p2, p3 = pl.pallas_call(
        _att_body,
        grid=(NB,),
        in_specs=[
            pl.BlockSpec((B, D), lambda i: (i, 0)),
            pl.BlockSpec((B, D), lambda i: (i, 0)),
            _full((D, D)),
            _full((D, D)), _full((D, D)), _full((D, D)),
        ],
        out_specs=[pl.BlockSpec((B, D), lambda i: (i, 0))] * 5
                  + [pl.BlockSpec((1, 1, D), lambda i: (i, 0, 0))] * 4,
        out_shape=[jax.ShapeDtypeStruct((N, D), jnp.float32)] * 5
                  + [jax.ShapeDtypeStruct((NB, 1, D), jnp.float32)] * 4,
    )(x, spa, A, W_poi, Whead_p, Wdec_p)

    cla = jnp.stack([jnp.sum(p0[:, 0], axis=0), jnp.sum(p1[:, 0], axis=0),
                     jnp.sum(p2[:, 0], axis=0), jnp.sum(p3[:, 0], axis=0)])[:, :C] * (1.0 / N)
    cam = jnp.stack([c0, c1, c2, c3])[:, :, :C]
    out = out_p[:, :C]
    return (out, cla, cam)
